# baseline (speedup 1.0000x reference)
import functools

import jax
import jax.numpy as jnp
from jax import lax
from jax.experimental import pallas as pl
from jax.experimental.pallas import tpu as pltpu
from jax.experimental.pallas import tpu_sc as plsc

NUM_WORKERS = 32
LANES = 16


def _build_sc_kernel(B, S, D, CHUNK):
    S_PER_W = S // NUM_WORKERS
    NCH = S_PER_W // CHUNK
    mesh = plsc.VectorSubcoreMesh(core_axis_name="c", subcore_axis_name="s")

    @functools.partial(
        pl.kernel,
        mesh=mesh,
        out_type=jax.ShapeDtypeStruct((B * (S + 1) * D,), jnp.float32),
        scratch_types=[
            pltpu.VMEM((B, S_PER_W), jnp.int32),
            pltpu.VMEM((S_PER_W, D), jnp.float32),
            pltpu.VMEM((CHUNK, D), jnp.float32),
            pltpu.VMEM((CHUNK * D,), jnp.float32),
            pltpu.VMEM((D,), jnp.float32),
            pltpu.SemaphoreType.DMA,
        ],
    )
    def sc_embed(idx_hbm, table_hbm, pos_hbm, cls_hbm, out_hbm,
                 idx_v, pos_v, rows_v, sum_v, cls_v, sem):
        wid = lax.axis_index("c") * 16 + lax.axis_index("s")
        s0 = wid * S_PER_W

        pltpu.sync_copy(pos_hbm.at[pl.ds(s0, S_PER_W)], pos_v)

        for b in range(B):
            pltpu.sync_copy(idx_hbm.at[pl.ds(b * S + s0, S_PER_W)],
                            idx_v.at[b])

        @pl.when(wid < B)
        def _():
            pltpu.sync_copy(cls_hbm, cls_v)
            pltpu.sync_copy(cls_v, out_hbm.at[pl.ds(wid * (S + 1) * D, D)])

        for b in range(B):
            out_base = (b * (S + 1) + 1 + s0) * D

            @pl.loop(0, NCH)
            def _(c):
                pltpu.async_copy(
                    table_hbm.at[idx_v.at[b, pl.ds(c * CHUNK, CHUNK)]],
                    rows_v, sem).wait()

                @pl.loop(0, CHUNK)
                def _(r):
                    @pl.loop(0, D, step=LANES)
                    def _(l):
                        sum_v[pl.ds(r * D + l, LANES)] = (
                            rows_v[r, pl.ds(l, LANES)]
                            + pos_v[c * CHUNK + r, pl.ds(l, LANES)])

                pltpu.sync_copy(
                    sum_v, out_hbm.at[pl.ds(out_base + c * CHUNK * D,
                                            CHUNK * D)])

    return sc_embed


def kernel(input, table, pos_embeds, cls):
    B, S = input.shape
    D = table.shape[1]
    idx_flat = input.reshape(B * S)
    pos2d = pos_embeds.reshape(S, D)
    cls1d = cls.reshape(D)
    sc = _build_sc_kernel(B, S, D, CHUNK=16)
    out_flat = sc(idx_flat, table, pos2d, cls1d)
    return out_flat.reshape(B, S + 1, D)


# speedup vs baseline: 1.1631x; 1.1631x over previous
import functools

import jax
import jax.numpy as jnp
from jax import lax
from jax.experimental import pallas as pl
from jax.experimental.pallas import tpu as pltpu
from jax.experimental.pallas import tpu_sc as plsc

NUM_WORKERS = 32
LANES = 16


def _build_sc_kernel(B, S, D, CHUNK):
    S_PER_W = S // NUM_WORKERS
    NCH = S_PER_W // CHUNK
    T = NCH * B
    mesh = plsc.VectorSubcoreMesh(core_axis_name="c", subcore_axis_name="s")

    @functools.partial(
        pl.kernel,
        mesh=mesh,
        out_type=jax.ShapeDtypeStruct((B * (S + 1) * D,), jnp.float32),
        scratch_types=[
            pltpu.VMEM((B, S_PER_W), jnp.int32),
            pltpu.VMEM((CHUNK, D), jnp.float32),
            pltpu.VMEM((CHUNK, D), jnp.float32),
            pltpu.VMEM((CHUNK, D), jnp.float32),
            pltpu.VMEM((CHUNK * D,), jnp.float32),
            pltpu.VMEM((CHUNK * D,), jnp.float32),
            pltpu.VMEM((D,), jnp.float32),
            pltpu.SemaphoreType.DMA,
            pltpu.SemaphoreType.DMA,
            pltpu.SemaphoreType.DMA,
            pltpu.SemaphoreType.DMA,
        ],
    )
    def sc_embed(idx_hbm, table_hbm, pos_hbm, cls_hbm, out_hbm,
                 idx_v, pos_c, rows0, rows1, sum0, sum1, cls_v,
                 sg0, sg1, so0, so1):
        wid = lax.axis_index("c") * 16 + lax.axis_index("s")
        s0 = wid * S_PER_W
        rows = (rows0, rows1)
        sums = (sum0, sum1)
        sgs = (sg0, sg1)
        sos = (so0, so1)

        for b in range(B):
            pltpu.sync_copy(idx_hbm.at[pl.ds(b * S + s0, S_PER_W)],
                            idx_v.at[b])

        @pl.when(wid < B)
        def _():
            pltpu.sync_copy(cls_hbm, cls_v)
            pltpu.sync_copy(cls_v, out_hbm.at[pl.ds(wid * (S + 1) * D, D)])

        def gather_start(t, k):
            b = t % B
            c = t // B
            pltpu.async_copy(
                table_hbm.at[idx_v.at[b, pl.ds(c * CHUNK, CHUNK)]],
                rows[k], sgs[k])

        def gather_wait(k):
            pltpu.make_async_copy(table_hbm.at[pl.ds(0, CHUNK)],
                                  rows[k], sgs[k]).wait()

        def out_start(t, k):
            b = t % B
            c = t // B
            off = (b * (S + 1) + 1 + s0 + c * CHUNK) * D
            pltpu.async_copy(sums[k], out_hbm.at[pl.ds(off, CHUNK * D)],
                             sos[k])

        def out_wait(k):
            pltpu.make_async_copy(sums[k], out_hbm.at[pl.ds(0, CHUNK * D)],
                                  sos[k]).wait()

        gather_start(0, 0)

        @pl.loop(0, T, step=2)
        def _(tt):
            for kk in range(2):
                t = tt + kk

                @pl.when(t % B == 0)
                def _():
                    c = t // B
                    pltpu.sync_copy(
                        pos_hbm.at[pl.ds(s0 + c * CHUNK, CHUNK)], pos_c)

                @pl.when(t + 1 < T)
                def _():
                    gather_start(t + 1, 1 - kk)

                gather_wait(kk)

                @pl.when(t >= 2)
                def _():
                    out_wait(kk)

                @pl.loop(0, CHUNK)
                def _(r):
                    for l in range(0, D, LANES):
                        sums[kk][pl.ds(r * D + l, LANES)] = (
                            rows[kk][r, pl.ds(l, LANES)]
                            + pos_c[r, pl.ds(l, LANES)])

                out_start(t, kk)

        out_wait(0)
        out_wait(1)

    return sc_embed


def kernel(input, table, pos_embeds, cls):
    B, S = input.shape
    D = table.shape[1]
    idx_flat = input.reshape(B * S)
    pos2d = pos_embeds.reshape(S, D)
    cls1d = cls.reshape(D)
    sc = _build_sc_kernel(B, S, D, CHUNK=16)
    out_flat = sc(idx_flat, table, pos2d, cls1d)
    return out_flat.reshape(B, S + 1, D)


# speedup vs baseline: 1.4709x; 1.2646x over previous
import functools

import jax
import jax.numpy as jnp
from jax import lax
from jax.experimental import pallas as pl
from jax.experimental.pallas import tpu as pltpu
from jax.experimental.pallas import tpu_sc as plsc

NUM_WORKERS = 32
LANES = 16


def _build_sc_kernel(B, S, D, CHUNK):
    SP = S + 1
    P = ((SP + 7) // 8) * 8
    S_PER_W = S // NUM_WORKERS
    NCH = S_PER_W // CHUNK
    T = NCH * B
    mesh = plsc.VectorSubcoreMesh(core_axis_name="c", subcore_axis_name="s")

    @functools.partial(
        pl.kernel,
        mesh=mesh,
        out_type=jax.ShapeDtypeStruct((B, SP, D), jnp.float32),
        scratch_types=[
            pltpu.VMEM((B, S_PER_W), jnp.int32),
            pltpu.VMEM((CHUNK,), jnp.int32),
            pltpu.VMEM((CHUNK, D), jnp.float32),
            pltpu.VMEM((CHUNK, D), jnp.float32),
            pltpu.VMEM((CHUNK, D), jnp.float32),
            pltpu.VMEM((D,), jnp.float32),
            pltpu.SemaphoreType.DMA,
            pltpu.SemaphoreType.DMA,
            pltpu.SemaphoreType.DMA,
            pltpu.SemaphoreType.DMA,
            pltpu.SemaphoreType.DMA,
        ],
    )
    def sc_embed(sidx_hbm, pidx_hbm, table_hbm, pos_hbm, cls_hbm, out_hbm,
                 idx_v, pidx_v, pos_c, rows0, rows1, cls_v,
                 sg0, sg1, so0, so1, sp):
        wid = lax.axis_index("c") * 16 + lax.axis_index("s")
        s0 = wid * S_PER_W
        rows = (rows0, rows1)
        sgs = (sg0, sg1)
        sos = (so0, so1)

        for b in range(B):
            pltpu.sync_copy(sidx_hbm.at[pl.ds(b * P + s0, S_PER_W)],
                            idx_v.at[b])

        @pl.when(wid == 0)
        def _():
            pltpu.sync_copy(cls_hbm, cls_v)

        def gather_start(t, k):
            b = t % B
            c = t // B
            pltpu.async_copy(
                table_hbm.at[idx_v.at[b, pl.ds(c * CHUNK, CHUNK)]],
                rows[k], sgs[k])

        def gather_wait(k):
            pltpu.make_async_copy(table_hbm.at[pl.ds(0, CHUNK)],
                                  rows[k], sgs[k]).wait()

        def out_start(t, k):
            b = t % B
            g = s0 + (t // B) * CHUNK
            pltpu.async_copy(rows[k], out_hbm.at[b, pl.ds(g, CHUNK)], sos[k])

        def out_wait(k):
            pltpu.make_async_copy(rows[k], out_hbm.at[0, pl.ds(0, CHUNK)],
                                  sos[k]).wait()

        gather_start(0, 0)

        @pl.loop(0, T, step=2)
        def _(tt):
            for kk in range(2):
                t = tt + kk

                @pl.when(t % B == 0)
                def _():
                    g = s0 + (t // B) * CHUNK
                    pltpu.sync_copy(pidx_hbm.at[pl.ds(g, CHUNK)], pidx_v)
                    pltpu.async_copy(pos_hbm.at[pidx_v], pos_c, sp).wait()

                @pl.when(t + 1 < T)
                def _():
                    @pl.when(t >= 1)
                    def _():
                        out_wait(1 - kk)
                    gather_start(t + 1, 1 - kk)

                gather_wait(kk)

                @pl.loop(0, CHUNK, step=2)
                def _(r):
                    for rr in range(2):
                        for l in range(0, D, LANES):
                            rows[kk][r + rr, pl.ds(l, LANES)] = (
                                rows[kk][r + rr, pl.ds(l, LANES)]
                                + pos_c[r + rr, pl.ds(l, LANES)])

                @pl.when((wid == 0) & (t < B))
                def _():
                    for l in range(0, D, LANES):
                        rows[kk][0, pl.ds(l, LANES)] = cls_v[pl.ds(l, LANES)]

                out_start(t, kk)

        out_wait(0)
        out_wait(1)

        @pl.when(wid == NUM_WORKERS - 1)
        def _():
            for b in range(B):
                pltpu.sync_copy(sidx_hbm.at[pl.ds(b * P + S, 8)],
                                idx_v.at[0, pl.ds(0, 8)])
                pltpu.async_copy(
                    table_hbm.at[idx_v.at[0, pl.ds(0, 8)]],
                    rows0.at[pl.ds(0, 8)], sg0).wait()
                pltpu.sync_copy(pidx_hbm.at[pl.ds(S, 8)],
                                pidx_v.at[pl.ds(0, 8)])
                pltpu.async_copy(pos_hbm.at[pidx_v.at[pl.ds(0, 8)]],
                                 pos_c.at[pl.ds(0, 8)], sp).wait()
                for l in range(0, D, LANES):
                    rows0[0, pl.ds(l, LANES)] = (
                        rows0[0, pl.ds(l, LANES)] + pos_c[0, pl.ds(l, LANES)])
                pltpu.sync_copy(rows0.at[pl.ds(0, 1)],
                                out_hbm.at[b, pl.ds(S, 1)])

    return sc_embed


def kernel(input, table, pos_embeds, cls):
    B, S = input.shape
    D = table.shape[1]
    SP = S + 1
    P = ((SP + 7) // 8) * 8
    sidx = jnp.zeros((B, P), jnp.int32).at[:, 1:SP].set(input).reshape(B * P)
    pidx = jnp.zeros((P,), jnp.int32).at[1:SP].set(
        jnp.arange(S, dtype=jnp.int32))
    pos2d = pos_embeds.reshape(S, D)
    cls1d = cls.reshape(D)
    sc = _build_sc_kernel(B, S, D, CHUNK=32)
    return sc(sidx, pidx, table, pos2d, cls1d)


# speedup vs baseline: 2.9836x; 2.0284x over previous
import functools

import jax
import jax.numpy as jnp
from jax import lax
from jax.experimental import pallas as pl
from jax.experimental.pallas import tpu as pltpu
from jax.experimental.pallas import tpu_sc as plsc

NUM_WORKERS = 32
LANES = 16
CH = 4


def _build_sc_kernel(B, S, D, NB):
    SP = S + 1
    P = ((SP + 7) // 8) * 8
    S_PER_W = S // NUM_WORKERS
    T = S_PER_W // CH
    GI = B * CH
    SLAB = CH * B * D
    mesh = plsc.VectorSubcoreMesh(core_axis_name="c", subcore_axis_name="s")

    @functools.partial(
        pl.kernel,
        mesh=mesh,
        out_type=jax.ShapeDtypeStruct((SP * B * D,), jnp.float32),
        scratch_types=[
            pltpu.VMEM((T * GI + 8,), jnp.int32),
            pltpu.VMEM((T * 8 + 8,), jnp.int32),
            pltpu.VMEM((GI, D), jnp.float32),
            pltpu.VMEM((GI, D), jnp.float32),
            pltpu.VMEM((CH, D), jnp.float32),
            pltpu.VMEM((CH, D), jnp.float32),
            pltpu.VMEM((SLAB,), jnp.float32),
            pltpu.VMEM((SLAB,), jnp.float32),
            pltpu.VMEM((D,), jnp.float32),
            pltpu.SemaphoreType.DMA,
            pltpu.SemaphoreType.DMA,
            pltpu.SemaphoreType.DMA,
            pltpu.SemaphoreType.DMA,
            pltpu.SemaphoreType.DMA,
            pltpu.SemaphoreType.DMA,
        ],
    )
    def sc_embed(gidx_hbm, pidx_hbm, table_hbm, pos_hbm, cls_hbm, out_hbm,
                 gidx_v, pidx_v, rows0, rows1, posb0, posb1, slab0, slab1,
                 cls_v, sg0, sg1, sp0, sp1, so0, so1):
        wid = lax.axis_index("c") * 16 + lax.axis_index("s")
        s0 = wid * S_PER_W
        rows = (rows0, rows1)
        posb = (posb0, posb1)
        slab = (slab0, slab1)
        sgs = (sg0, sg1)
        sps = (sp0, sp1)
        sos = (so0, so1)

        pltpu.sync_copy(gidx_hbm.at[pl.ds(wid * T * GI, T * GI)],
                        gidx_v.at[pl.ds(0, T * GI)])
        pltpu.sync_copy(pidx_hbm.at[pl.ds(wid * T * 8, T * 8)],
                        pidx_v.at[pl.ds(0, T * 8)])

        @pl.when(wid == 0)
        def _():
            pltpu.sync_copy(cls_hbm, cls_v)

        def gather_start(t, k):
            pltpu.async_copy(
                table_hbm.at[gidx_v.at[pl.ds(t * GI, GI)]], rows[k], sgs[k])
            pltpu.async_copy(
                pos_hbm.at[pidx_v.at[pl.ds(t * 8, CH)]], posb[k], sps[k])

        def gather_wait(k):
            pltpu.make_async_copy(table_hbm.at[pl.ds(0, GI)],
                                  rows[k], sgs[k]).wait()
            pltpu.make_async_copy(pos_hbm.at[pl.ds(0, CH)],
                                  posb[k], sps[k]).wait()

        def out_start(t, k):
            off = (s0 + t * CH) * B * D
            pltpu.async_copy(slab[k], out_hbm.at[pl.ds(off, SLAB)], sos[k])

        def out_wait(k):
            pltpu.make_async_copy(slab[k], out_hbm.at[pl.ds(0, SLAB)],
                                  sos[k]).wait()

        def add_interleave(k, n_r):
            for r in range(n_r):
                for lb in range(NB):
                    for v in range(128 // LANES):
                        l = lb * 128 + v * LANES
                        pv = posb[k][r, pl.ds(l, LANES)]
                        for b in range(B):
                            o = r * B * D + lb * B * 128 + b * 128 + v * LANES
                            slab[k][pl.ds(o, LANES)] = (
                                rows[k][b * CH + r, pl.ds(l, LANES)] + pv)

        gather_start(0, 0)

        @pl.loop(0, T, step=2)
        def _(tt):
            for kk in range(2):
                t = tt + kk

                @pl.when(t + 1 < T)
                def _():
                    gather_start(t + 1, 1 - kk)

                gather_wait(kk)

                @pl.when(t >= 2)
                def _():
                    out_wait(kk)

                add_interleave(kk, CH)

                if kk == 0:
                    @pl.when((wid == 0) & (t == 0))
                    def _():
                        for lb in range(NB):
                            for v in range(128 // LANES):
                                l = lb * 128 + v * LANES
                                cv = cls_v[pl.ds(l, LANES)]
                                for b in range(B):
                                    o = lb * B * 128 + b * 128 + v * LANES
                                    slab[kk][pl.ds(o, LANES)] = cv

                out_start(t, kk)

        out_wait(0)
        out_wait(1)

        @pl.when(wid == NUM_WORKERS - 1)
        def _():
            pltpu.sync_copy(gidx_hbm.at[pl.ds(NUM_WORKERS * T * GI, 8)],
                            gidx_v.at[pl.ds(0, 8)])
            pltpu.async_copy(table_hbm.at[gidx_v.at[pl.ds(0, 8)]],
                             rows0.at[pl.ds(0, 8)], sg0).wait()
            pltpu.sync_copy(pidx_hbm.at[pl.ds(NUM_WORKERS * T * 8, 8)],
                            pidx_v.at[pl.ds(0, 8)])
            pltpu.async_copy(pos_hbm.at[pidx_v.at[pl.ds(0, CH)]],
                             posb0, sp0).wait()
            for lb in range(NB):
                for v in range(128 // LANES):
                    l = lb * 128 + v * LANES
                    pv = posb0[0, pl.ds(l, LANES)]
                    for b in range(B):
                        o = lb * B * 128 + b * 128 + v * LANES
                        slab0[pl.ds(o, LANES)] = (
                            rows0[b, pl.ds(l, LANES)] + pv)
            pltpu.sync_copy(slab0.at[pl.ds(0, B * D)],
                            out_hbm.at[pl.ds(S * B * D, B * D)])

    return sc_embed


def kernel(input, table, pos_embeds, cls):
    B, S = input.shape
    D = table.shape[1]
    SP = S + 1
    P = ((SP + 7) // 8) * 8
    NB = D // 128
    S_PER_W = S // NUM_WORKERS
    sidx = jnp.zeros((B, SP), jnp.int32).at[:, 1:].set(input)
    gmain = (sidx[:, :S]
             .reshape(B, NUM_WORKERS, S_PER_W // CH, CH)
             .transpose(1, 2, 0, 3)
             .reshape(-1))
    gtail = jnp.concatenate([sidx[:, S], jnp.zeros((8 - B,), jnp.int32)])
    gidx = jnp.concatenate([gmain, gtail])
    n_items = S // CH
    pmain = jnp.clip(
        jnp.arange(n_items, dtype=jnp.int32)[:, None] * CH
        + jnp.arange(8, dtype=jnp.int32)[None, :] - 1,
        0, S - 1).reshape(-1)
    ptail = jnp.full((8,), S - 1, jnp.int32)
    pidx = jnp.concatenate([pmain, ptail])
    pos2d = pos_embeds.reshape(S, D)
    cls1d = cls.reshape(D)
    sc = _build_sc_kernel(B, S, D, NB)
    out_flat = sc(gidx, pidx, table, pos2d, cls1d)
    return (out_flat.reshape(SP, NB, B, 128)
            .transpose(2, 0, 1, 3)
            .reshape(B, SP, D))
